# baseline (speedup 1.0000x reference)
import jax
import jax.numpy as jnp
from jax import lax
from jax.experimental import pallas as pl
from jax.experimental.pallas import tpu as pltpu
from jax.experimental.pallas import tpu_sc as plsc

_NC = 2
_NS = 16
_NW = _NC * _NS
_L = 16

_B = 16384
_D = 64
_CW = 2 * _D
_BPW = _B // _NW
_CH = 128
_NCH = _BPW // _CH
_FROWS = 112
_ZROW = 104


def _fused_body(wl_ref, wc_ref, n0_ref, o_ref):
    k4 = lax.broadcasted_iota(jnp.int32, (_FROWS, 4), 0)
    j4 = lax.broadcasted_iota(jnp.int32, (_FROWS, 4), 1)
    a = ((k4 // 26) == j4).astype(jnp.float32)
    k26 = lax.broadcasted_iota(jnp.int32, (_FROWS, 26), 0)
    j26 = lax.broadcasted_iota(jnp.int32, (_FROWS, 26), 1)
    b = (((k26 % 26) == j26) & (k26 < 104)).astype(jnp.float32)
    live = (lax.broadcasted_iota(jnp.int32, (_FROWS, 1), 0) < 104)
    left = (
        jnp.dot(a, wl_ref[...], preferred_element_type=jnp.float32,
                precision=lax.Precision.HIGHEST)
        + jnp.dot(b, wc_ref[...], preferred_element_type=jnp.float32,
                  precision=lax.Precision.HIGHEST)
        + jnp.where(live, n0_ref[...], 0.0)
    )
    o_ref[...] = jnp.concatenate(
        [left, jnp.zeros((_FROWS, _D), jnp.float32)], axis=1)


def _build_fused(wl, wc, n0):
    return pl.pallas_call(
        _fused_body,
        out_shape=jax.ShapeDtypeStruct((_FROWS, _CW), jnp.float32),
    )(wl, wc, n0)


def _main_body(c_hbm, idx_hbm, nu_hbm, fu_hbm, out_hbm,
               idx_v, nu_v, g1_v, nm_v, c_v, fu_v, out_v, sem):
    wid = lax.axis_index("s") * _NC + lax.axis_index("c")
    base = wid * _BPW

    with jax.named_scope("stage"):
        pltpu.sync_copy(idx_hbm.at[pl.ds(base, _BPW)], idx_v)
        pltpu.sync_copy(nu_hbm, nu_v)
        fdesc = pltpu.async_copy(fu_hbm, fu_v, sem)

    nu = nu_v[...]
    with jax.named_scope("pass1"):
        for i in range(_BPW // _L):
            sl = pl.ds(i * _L, _L)
            idxc = idx_v[sl]
            item = idxc >= nu
            g1_v[sl] = jnp.where(item, idxc + 30, idxc)
            nm_v[sl] = jnp.where(item, 1.0, 0.0)
        fdesc.wait()

    for h in range(2):
        hof = h * (_BPW // 2)
        with jax.named_scope("gather"):
            descs = []
            for j in range(_NCH // 2):
                sl = pl.ds(hof + j * _CH, _CH)
                dl = pl.ds(j * _CH, _CH)
                descs.append(
                    pltpu.async_copy(c_hbm.at[g1_v.at[sl]], c_v.at[dl], sem))
            for d in descs:
                d.wait()

        with jax.named_scope("combine"):
            @plsc.parallel_loop(0, _BPW // (2 * _L))
            def _combine(ci):
                nm16 = nm_v[pl.ds(hof + ci * _L, _L)]
                for k in range(_L):
                    bb = ci * _L + k
                    nm_s = nm16[k]
                    tv = c_v[bb, pl.ds(_D, _L)]
                    lvl = tv[0].astype(jnp.int32)
                    ins = tv[1].astype(jnp.int32)
                    fr_s = jnp.where(nm_s > 0.5, _ZROW, lvl * 26 + ins)
                    for j in range(_D // _L):
                        sl = pl.ds(j * _L, _L)
                        out_v[bb, sl] = (
                            c_v[bb, sl]
                            + nm_s * c_v[bb, pl.ds(_D + j * _L, _L)]
                            + fu_v[fr_s, sl]
                        )

        with jax.named_scope("writeback"):
            pltpu.sync_copy(out_v, out_hbm.at[pl.ds(base + hof, _BPW // 2)])


def _sc_main(ctab, idx, nu_vec, fused):
    mesh = plsc.VectorSubcoreMesh(core_axis_name="c", subcore_axis_name="s")
    return pl.kernel(
        _main_body,
        out_type=jax.ShapeDtypeStruct((_B, _D), jnp.float32),
        mesh=mesh,
        scratch_types=[
            pltpu.VMEM((_BPW,), jnp.int32),
            pltpu.VMEM((_L,), jnp.int32),
            pltpu.VMEM((_BPW,), jnp.int32),
            pltpu.VMEM((_BPW,), jnp.float32),
            pltpu.VMEM((_BPW // 2, _CW), jnp.float32),
            pltpu.VMEM((_FROWS, _CW), jnp.float32),
            pltpu.VMEM((_BPW // 2, _D), jnp.float32),
            pltpu.SemaphoreType.DMA,
        ],
    )(ctab, idx, nu_vec, fused)


@jax.jit
def _run(ctab, idx, nu_vec, fused):
    return _sc_main(ctab, idx, nu_vec, fused)


def kernel(x, idx, num_users, weight, name_emb):
    x = x.astype(jnp.int32)
    idx = idx.astype(jnp.int32)
    nu = jnp.asarray(num_users, jnp.int32)
    wl = lax.dynamic_slice_in_dim(weight, nu, 4, axis=0)
    wc = lax.dynamic_slice_in_dim(weight, nu + 4, 26, axis=0)
    fused = _build_fused(wl, wc, name_emb[0:1])
    nrows = weight.shape[0]
    xpad = jnp.pad(x[:, 1:3].astype(jnp.float32),
                   ((0, nrows - x.shape[0]), (0, _D - 2)))
    row = lax.broadcasted_iota(jnp.int32, (nrows, 1), 0)
    right = jnp.where(row < nu, xpad, name_emb)
    ctab = jnp.concatenate([weight, right], axis=1)
    nu_vec = jnp.full((_L,), nu, jnp.int32)
    return _run(ctab, idx, nu_vec, fused)


# speedup vs baseline: 1.4461x; 1.4461x over previous
import jax
import jax.numpy as jnp
from jax import lax
from jax.experimental import pallas as pl
from jax.experimental.pallas import tpu as pltpu
from jax.experimental.pallas import tpu_sc as plsc

_NC = 2
_NS = 16
_NW = _NC * _NS
_L = 16

_B = 16384
_D = 64
_CW = 2 * _D
_BPW = _B // _NW
_CH = 128
_NCH = _BPW // _CH
_FROWS = 112
_ZROW = 104


def _fused_body(wl_ref, wc_ref, n0_ref, o_ref):
    k4 = lax.broadcasted_iota(jnp.int32, (_FROWS, 4), 0)
    j4 = lax.broadcasted_iota(jnp.int32, (_FROWS, 4), 1)
    a = ((k4 // 26) == j4).astype(jnp.float32)
    k26 = lax.broadcasted_iota(jnp.int32, (_FROWS, 26), 0)
    j26 = lax.broadcasted_iota(jnp.int32, (_FROWS, 26), 1)
    b = (((k26 % 26) == j26) & (k26 < 104)).astype(jnp.float32)
    live = (lax.broadcasted_iota(jnp.int32, (_FROWS, 1), 0) < 104)
    left = (
        jnp.dot(a, wl_ref[...], preferred_element_type=jnp.float32,
                precision=lax.Precision.HIGHEST)
        + jnp.dot(b, wc_ref[...], preferred_element_type=jnp.float32,
                  precision=lax.Precision.HIGHEST)
        + jnp.where(live, n0_ref[...], 0.0)
    )
    o_ref[...] = jnp.concatenate(
        [left, jnp.zeros((_FROWS, _D), jnp.float32)], axis=1)


def _build_fused(wl, wc, n0):
    return pl.pallas_call(
        _fused_body,
        out_shape=jax.ShapeDtypeStruct((_FROWS, _CW), jnp.float32),
    )(wl, wc, n0)


def _main_body(c_hbm, x1_hbm, x2_hbm, idx_hbm, nu_hbm, fu_hbm, out_hbm,
               idx_v, nu_v, g1_v, nm_v, lv_v, in_v, fr_v,
               c_v, fu_v, out_v, sem, sem2):
    wid = lax.axis_index("s") * _NC + lax.axis_index("c")
    base = wid * _BPW

    with jax.named_scope("stage"):
        pltpu.sync_copy(idx_hbm.at[pl.ds(base, _BPW)], idx_v)
        pltpu.sync_copy(nu_hbm, nu_v)
        fdesc = pltpu.async_copy(fu_hbm, fu_v, sem)

    with jax.named_scope("gather_x"):
        xdescs = []
        for j in range(_NCH):
            sl = pl.ds(j * _CH, _CH)
            xdescs.append(
                pltpu.async_copy(x1_hbm.at[idx_v.at[sl]], lv_v.at[sl], sem2))
            xdescs.append(
                pltpu.async_copy(x2_hbm.at[idx_v.at[sl]], in_v.at[sl], sem2))

    nu = nu_v[...]
    with jax.named_scope("pass1"):
        for i in range(_BPW // _L):
            sl = pl.ds(i * _L, _L)
            idxc = idx_v[sl]
            item = idxc >= nu
            g1_v[sl] = jnp.where(item, idxc + 30, idxc)
            nm_v[sl] = jnp.where(item, 1.0, 0.0)

    with jax.named_scope("pass2"):
        for d in xdescs:
            d.wait()
        for i in range(_BPW // _L):
            sl = pl.ds(i * _L, _L)
            fr_v[sl] = jnp.where(idx_v[sl] >= nu, _ZROW,
                                 lv_v[sl] * 26 + in_v[sl])
        fdesc.wait()

    for h in range(2):
        hof = h * (_BPW // 2)
        with jax.named_scope("gather"):
            descs = []
            for j in range(_NCH // 2):
                sl = pl.ds(hof + j * _CH, _CH)
                dl = pl.ds(j * _CH, _CH)
                descs.append(
                    pltpu.async_copy(c_hbm.at[g1_v.at[sl]], c_v.at[dl], sem))
            for d in descs:
                d.wait()

        with jax.named_scope("combine"):
            @plsc.parallel_loop(0, _BPW // (2 * _L))
            def _combine(ci):
                fr16 = fr_v[pl.ds(hof + ci * _L, _L)]
                nm16 = nm_v[pl.ds(hof + ci * _L, _L)]
                for k in range(_L):
                    bb = ci * _L + k
                    fr_s = fr16[k]
                    nm_s = nm16[k]
                    for j in range(_D // _L):
                        sl = pl.ds(j * _L, _L)
                        out_v[bb, sl] = (
                            c_v[bb, sl]
                            + nm_s * c_v[bb, pl.ds(_D + j * _L, _L)]
                            + fu_v[fr_s, sl]
                        )

        with jax.named_scope("writeback"):
            pltpu.sync_copy(out_v, out_hbm.at[pl.ds(base + hof, _BPW // 2)])


def _sc_main(ctab, x1, x2, idx, nu_vec, fused):
    mesh = plsc.VectorSubcoreMesh(core_axis_name="c", subcore_axis_name="s")
    return pl.kernel(
        _main_body,
        out_type=jax.ShapeDtypeStruct((_B, _D), jnp.float32),
        mesh=mesh,
        scratch_types=[
            pltpu.VMEM((_BPW,), jnp.int32),
            pltpu.VMEM((_L,), jnp.int32),
            pltpu.VMEM((_BPW,), jnp.int32),
            pltpu.VMEM((_BPW,), jnp.float32),
            pltpu.VMEM((_BPW,), jnp.int32),
            pltpu.VMEM((_BPW,), jnp.int32),
            pltpu.VMEM((_BPW,), jnp.int32),
            pltpu.VMEM((_BPW // 2, _CW), jnp.float32),
            pltpu.VMEM((_FROWS, _CW), jnp.float32),
            pltpu.VMEM((_BPW // 2, _D), jnp.float32),
            pltpu.SemaphoreType.DMA,
            pltpu.SemaphoreType.DMA,
        ],
    )(ctab, x1, x2, idx, nu_vec, fused)


@jax.jit
def _run(ctab, x1, x2, idx, nu_vec, fused):
    return _sc_main(ctab, x1, x2, idx, nu_vec, fused)


def kernel(x, idx, num_users, weight, name_emb):
    x = x.astype(jnp.int32)
    idx = idx.astype(jnp.int32)
    nu = jnp.asarray(num_users, jnp.int32)
    wl = lax.dynamic_slice_in_dim(weight, nu, 4, axis=0)
    wc = lax.dynamic_slice_in_dim(weight, nu + 4, 26, axis=0)
    fused = _build_fused(wl, wc, name_emb[0:1])
    ctab = jnp.concatenate([weight, name_emb], axis=1)
    x1 = jnp.asarray(x[:, 1])
    x2 = jnp.asarray(x[:, 2])
    nu_vec = jnp.full((_L,), nu, jnp.int32)
    return _run(ctab, x1, x2, idx, nu_vec, fused)
